# Initial kernel scaffold; baseline (speedup 1.0000x reference)
import jax
import jax.numpy as jnp
from jax.experimental import pallas as pl


def kernel(hidden_states, gate_weight, gate_up_proj, down_proj):
    raise NotImplementedError("write your pallas kernel here")


# speedup vs baseline: 1.4536x; 1.4536x over previous
import jax
import jax.numpy as jnp
from jax.experimental import pallas as pl

_E = 16
_TOPK = 4
_DFF = 768


def _router_body(x_ref, gw_ref, w_ref):
    x = x_ref[...]
    logits = jax.lax.dot_general(
        x, gw_ref[...], (((1,), (1,)), ((), ())),
        preferred_element_type=jnp.float32)
    m = jnp.max(logits, axis=-1, keepdims=True)
    ex = jnp.exp(logits - m)
    probs = ex / jnp.sum(ex, axis=-1, keepdims=True)
    p = probs
    sel = jnp.zeros_like(probs)
    idx = jax.lax.broadcasted_iota(jnp.int32, probs.shape, 1)
    for _ in range(_TOPK):
        cur = jnp.max(p, axis=-1, keepdims=True)
        amax = jnp.min(jnp.where(p == cur, idx, _E), axis=-1, keepdims=True)
        mask = idx == amax
        sel = jnp.where(mask, probs, sel)
        p = jnp.where(mask, -jnp.inf, p)
    denom = jnp.clip(jnp.sum(sel, axis=-1, keepdims=True), 1e-12, None)
    w = sel / denom
    w_ref[...] = jnp.transpose(w, (1, 0)).reshape(_E, 1, w.shape[0])


def _moe_body(w_ref, x_ref, gup_ref, down_ref, out_ref):
    e = pl.program_id(1)

    @pl.when(e == 0)
    def _():
        out_ref[...] = jnp.zeros_like(out_ref)

    x = x_ref[...]
    gu = jax.lax.dot_general(
        x, gup_ref[0], (((1,), (1,)), ((), ())),
        preferred_element_type=jnp.float32)
    gate = gu[:, :_DFF]
    up = gu[:, _DFF:]
    h = (gate * jax.lax.logistic(gate) * up).astype(jnp.bfloat16)
    dout = jax.lax.dot_general(
        h, down_ref[0], (((1,), (1,)), ((), ())),
        preferred_element_type=jnp.float32)
    out_ref[...] += w_ref[0, 0, :][:, None] * dout


def kernel(hidden_states, gate_weight, gate_up_proj, down_proj):
    b, s, d = hidden_states.shape
    x = hidden_states.reshape(s, d)

    w = pl.pallas_call(
        _router_body,
        out_shape=jax.ShapeDtypeStruct((_E, 1, s), jnp.float32),
    )(x, gate_weight)

    xb = x.astype(jnp.bfloat16)
    gupb = gate_up_proj.astype(jnp.bfloat16)
    downb = down_proj.astype(jnp.bfloat16)

    nt = 2
    sb = s // nt
    out = pl.pallas_call(
        _moe_body,
        grid=(nt, _E),
        in_specs=[
            pl.BlockSpec((1, 1, sb), lambda t, e: (e, 0, t)),
            pl.BlockSpec((sb, d), lambda t, e: (t, 0)),
            pl.BlockSpec((1, 2 * _DFF, d), lambda t, e: (e, 0, 0)),
            pl.BlockSpec((1, d, _DFF), lambda t, e: (e, 0, 0)),
        ],
        out_specs=pl.BlockSpec((sb, d), lambda t, e: (t, 0)),
        out_shape=jax.ShapeDtypeStruct((s, d), jnp.float32),
    )(w, xb, gupb, downb)
    return out.reshape(b, s, d)
